# baseline (speedup 1.0000x reference)
import functools

import jax
import jax.numpy as jnp
from jax import lax
import numpy as np
from jax.experimental import pallas as pl
from jax.experimental.pallas import tpu as pltpu

_INSIZE = 1024
_V = 100000
_S = 5
_B = 8
_CHUNK = 3584
_GRID = 14

_K0 = np.uint32(0)
_K1 = np.uint32(42)
_K2 = np.uint32(_K0 ^ _K1 ^ np.uint32(0x1BD11BDA))
_ROTS = ((13, 15, 26, 6), (17, 29, 16, 24))
_KS = (_K0, _K1, _K2)
_TINY = np.float32(np.finfo(np.float32).tiny)
_SPAN = np.float32(np.float32(1.0) - _TINY)


def _gumbel_from_counts(cnt):
    x0 = jnp.zeros_like(cnt) + _K0
    x1 = cnt + _K1
    for i in range(5):
        for r in _ROTS[i % 2]:
            x0 = x0 + x1
            x1 = (x1 << np.uint32(r)) | lax.shift_right_logical(
                x1, np.uint32(32 - r))
            x1 = x1 ^ x0
        x0 = x0 + _KS[(i + 1) % 3]
        x1 = x1 + _KS[(i + 2) % 3] + np.uint32(i + 1)
    bits = x0 ^ x1
    fb = lax.bitcast_convert_type(
        lax.shift_right_logical(bits, np.uint32(9)) | np.uint32(0x3F800000),
        jnp.float32) - np.float32(1.0)
    u = jnp.maximum(_TINY, fb * _SPAN + _TINY)
    return -jnp.log(-jnp.log(u))


def _block_logits(x, w, brow):
    return jax.lax.dot_general(
        x, w,
        dimension_numbers=(((1,), (1,)), ((), ())),
        preferred_element_type=jnp.float32,
    ) + brow


def _masked_gumbel_for_block(block_idx):
    v_iota = jax.lax.broadcasted_iota(jnp.int32, (_S, _B, _CHUNK), 2)
    s_iota = jax.lax.broadcasted_iota(jnp.int32, (_S, _B, _CHUNK), 0)
    b_iota = jax.lax.broadcasted_iota(jnp.int32, (_S, _B, _CHUNK), 1)
    gidx = block_idx * _CHUNK + v_iota
    cnt = (s_iota * (_B * _V) + b_iota * _V + gidx).astype(jnp.uint32)
    g = _gumbel_from_counts(cnt)
    return jnp.where(gidx < _V, g, -jnp.inf)


def _fused_kernel(x_ref, wa_ref, wb_ref, ba_ref, bb_ref, out_ref,
                  bestv, besti, gtail):
    c = pl.program_id(0)
    last = pl.num_programs(0) - 1

    @pl.when(c == 0)
    def _init():
        bestv[...] = jnp.full_like(bestv, -jnp.inf)
        besti[...] = jnp.zeros_like(besti)
        gtail[:, :, :_CHUNK] = _masked_gumbel_for_block(2 * last)
        gtail[:, :, _CHUNK:] = _masked_gumbel_for_block(2 * last + 1)

    x = x_ref[...]

    def _update(scores, blk):
        cmax = jnp.max(scores, axis=2)
        carg = jnp.argmax(scores, axis=2).astype(jnp.int32) + blk * _CHUNK
        better = cmax > bestv[...]
        besti[...] = jnp.where(better, carg, besti[...])
        bestv[...] = jnp.where(better, cmax, bestv[...])

    @pl.when(c < last)
    def _main():
        for w_ref, b_ref, blk in (
                (wa_ref, ba_ref, 2 * c),
                (wb_ref, bb_ref, 2 * c + 1)):
            logits = _block_logits(x, w_ref[...], b_ref[...])
            scores = _masked_gumbel_for_block(blk) + logits[None, :, :]
            _update(scores, blk)

    @pl.when(c == last)
    def _tail():
        for w_ref, b_ref, k, blk in (
                (wa_ref, ba_ref, 0, 2 * c),
                (wb_ref, bb_ref, 1, 2 * c + 1)):
            logits = _block_logits(x, w_ref[...], b_ref[...])
            g = gtail[:, :, k * _CHUNK:(k + 1) * _CHUNK]
            _update(g + logits[None, :, :], blk)
        out_ref[...] = besti[...]


@functools.partial(jax.jit, static_argnames=())
def kernel(x, W, b):
    b2 = b.reshape(1, _V)
    out = pl.pallas_call(
        _fused_kernel,
        grid=(_GRID,),
        in_specs=[
            pl.BlockSpec((_B, _INSIZE), lambda c: (0, 0)),
            pl.BlockSpec((_CHUNK, _INSIZE), lambda c: (2 * c, 0)),
            pl.BlockSpec((_CHUNK, _INSIZE), lambda c: (2 * c + 1, 0)),
            pl.BlockSpec((1, _CHUNK), lambda c: (0, 2 * c)),
            pl.BlockSpec((1, _CHUNK), lambda c: (0, 2 * c + 1)),
        ],
        out_specs=pl.BlockSpec((_S, _B), lambda c: (0, 0)),
        out_shape=jax.ShapeDtypeStruct((_S, _B), jnp.int32),
        scratch_shapes=[
            pltpu.VMEM((_S, _B), jnp.float32),
            pltpu.VMEM((_S, _B), jnp.int32),
            pltpu.VMEM((_S, _B, 2 * _CHUNK), jnp.float32),
        ],
        compiler_params=pltpu.CompilerParams(
            vmem_limit_bytes=100 * 1024 * 1024),
    )(x, W, W, b2, b2)
    return out


# speedup vs baseline: 1.0347x; 1.0347x over previous
import functools

import jax
import jax.numpy as jnp
from jax import lax
import numpy as np
from jax.experimental import pallas as pl
from jax.experimental.pallas import tpu as pltpu

_INSIZE = 1024
_V = 100000
_S = 5
_B = 8
_CHUNK = 3584
_GRID = 14

_K0 = np.uint32(0)
_K1 = np.uint32(42)
_K2 = np.uint32(_K0 ^ _K1 ^ np.uint32(0x1BD11BDA))
_ROTS = ((13, 15, 26, 6), (17, 29, 16, 24))
_KS = (_K0, _K1, _K2)
_TINY = np.float32(np.finfo(np.float32).tiny)
_SPAN = np.float32(np.float32(1.0) - _TINY)


def _gumbel_from_counts(cnt):
    x0 = jnp.zeros_like(cnt) + _K0
    x1 = cnt + _K1
    for i in range(5):
        for r in _ROTS[i % 2]:
            x0 = x0 + x1
            x1 = (x1 << np.uint32(r)) | lax.shift_right_logical(
                x1, np.uint32(32 - r))
            x1 = x1 ^ x0
        x0 = x0 + _KS[(i + 1) % 3]
        x1 = x1 + _KS[(i + 2) % 3] + np.uint32(i + 1)
    bits = x0 ^ x1
    fb = lax.bitcast_convert_type(
        lax.shift_right_logical(bits, np.uint32(9)) | np.uint32(0x3F800000),
        jnp.float32) - np.float32(1.0)
    u = jnp.maximum(_TINY, fb * _SPAN + _TINY)
    return -jnp.log(-jnp.log(u))


def _scores_for_block(x, w, brow, block_idx):
    logits = jax.lax.dot_general(
        x, w,
        dimension_numbers=(((1,), (1,)), ((), ())),
        preferred_element_type=jnp.float32,
    ) + brow

    v_iota = jax.lax.broadcasted_iota(jnp.int32, (_S, _B, _CHUNK), 2)
    s_iota = jax.lax.broadcasted_iota(jnp.int32, (_S, _B, _CHUNK), 0)
    b_iota = jax.lax.broadcasted_iota(jnp.int32, (_S, _B, _CHUNK), 1)
    gidx = block_idx * _CHUNK + v_iota
    cnt = (s_iota * (_B * _V) + b_iota * _V + gidx).astype(jnp.uint32)

    scores = _gumbel_from_counts(cnt) + logits[None, :, :]
    return jnp.where(gidx < _V, scores, -jnp.inf)


def _fused_kernel(x_ref, wa_ref, wb_ref, ba_ref, bb_ref, out_ref,
                  bestv, besti):
    c = pl.program_id(0)

    @pl.when(c == 0)
    def _init():
        bestv[...] = jnp.full_like(bestv, -jnp.inf)
        besti[...] = jnp.zeros_like(besti)

    x = x_ref[...]
    for w_ref, b_ref, blk in (
            (wa_ref, ba_ref, 2 * c),
            (wb_ref, bb_ref, 2 * c + 1)):
        scores = _scores_for_block(x, w_ref[...], b_ref[...], blk)
        cmax = jnp.max(scores, axis=2)
        carg = jnp.argmax(scores, axis=2).astype(jnp.int32) + blk * _CHUNK
        better = cmax > bestv[...]
        besti[...] = jnp.where(better, carg, besti[...])
        bestv[...] = jnp.where(better, cmax, bestv[...])

    @pl.when(c == pl.num_programs(0) - 1)
    def _fin():
        out_ref[...] = besti[...]


@functools.partial(jax.jit, static_argnames=())
def kernel(x, W, b):
    b2 = b.reshape(1, _V)
    out = pl.pallas_call(
        _fused_kernel,
        grid=(_GRID,),
        in_specs=[
            pl.BlockSpec((_B, _INSIZE), lambda c: (0, 0)),
            pl.BlockSpec((_CHUNK, _INSIZE), lambda c: (2 * c, 0)),
            pl.BlockSpec((_CHUNK, _INSIZE), lambda c: (2 * c + 1, 0)),
            pl.BlockSpec((1, _CHUNK), lambda c: (0, 2 * c)),
            pl.BlockSpec((1, _CHUNK), lambda c: (0, 2 * c + 1)),
        ],
        out_specs=pl.BlockSpec((_S, _B), lambda c: (0, 0)),
        out_shape=jax.ShapeDtypeStruct((_S, _B), jnp.int32),
        scratch_shapes=[
            pltpu.VMEM((_S, _B), jnp.float32),
            pltpu.VMEM((_S, _B), jnp.int32),
        ],
        compiler_params=pltpu.CompilerParams(
            vmem_limit_bytes=100 * 1024 * 1024),
    )(x, W, W, b2, b2)
    return out
